# baseline (speedup 1.0000x reference)
import functools

import jax
import jax.numpy as jnp
from jax import lax
from jax.experimental import pallas as pl
from jax.experimental.pallas import tpu as pltpu
from jax.experimental.pallas import tpu_sc as plsc

N_NODES = 10000
D_FEAT = 128
D_HID = 128
N_CLASSES = 40

NC = 2
NS = 16
NT = NC * NS
B = 128
NPAD = 10240
DUMMY = N_NODES
CH = NPAD // NS
NSC = 10112
CHS = NSC // NS
D2 = 128
BR = 512


def _wid():
    return lax.axis_index("s") * NC + lax.axis_index("c")


def _deg_body(nb, dst_hbm, ones_hbm, degp_hbm, idx_v, ones_v, deg_sh, sem):
    del sem
    c = lax.axis_index("c")
    s = lax.axis_index("s")
    pltpu.sync_copy(ones_hbm.at[pl.ds(s * CH, CH)], deg_sh.at[pl.ds(s * CH, CH)])
    pltpu.sync_copy(dst_hbm.at[pl.ds(_wid() * nb, nb)], idx_v)
    pltpu.sync_copy(ones_hbm.at[pl.ds(0, B)], ones_v)
    plsc.subcore_barrier()

    def step(j, carry):
        pltpu.sync_copy(ones_v, deg_sh.at[idx_v.at[j]], add=True)
        return carry

    lax.fori_loop(0, nb, step, 0)
    plsc.subcore_barrier()
    pltpu.sync_copy(deg_sh.at[pl.ds(s * CH, CH)],
                    degp_hbm.at[pl.ds(c * NPAD + s * CH, CH)])


def _make_deg(nb):
    return pl.kernel(
        functools.partial(_deg_body, nb),
        out_type=jax.ShapeDtypeStruct((NC * NPAD,), jnp.float32),
        mesh=plsc.VectorSubcoreMesh(core_axis_name="c", subcore_axis_name="s"),
        scratch_types=[
            pltpu.VMEM((nb, B), jnp.int32),
            pltpu.VMEM((B,), jnp.float32),
            pltpu.VMEM_SHARED((NPAD,), jnp.float32),
            pltpu.SemaphoreType.DMA,
        ],
    )


def _ring_copy(row_hbm, slot, sem0, sem1, par, wait):
    @pl.when(par == 0)
    def _():
        if wait:
            pltpu.make_async_copy(row_hbm, slot, sem0).wait()
        else:
            pltpu.async_copy(row_hbm, slot, sem0)

    @pl.when(par == 1)
    def _():
        if wait:
            pltpu.make_async_copy(row_hbm, slot, sem1).wait()
        else:
            pltpu.async_copy(row_hbm, slot, sem1)


def _agg_loop(nb, base, h_hbm, src_hbm, dst_hbm, sid4, did4, rows_v, agg_sh,
              gsem, ssem, si0, si1, di0, di1):
    pltpu.sync_copy(src_hbm.at[base], sid4.at[0])
    pltpu.sync_copy(dst_hbm.at[base], did4.at[0])

    @pl.when(nb > 1)
    def _():
        pltpu.sync_copy(src_hbm.at[base + 1], sid4.at[1])
        pltpu.sync_copy(dst_hbm.at[base + 1], did4.at[1])

    pltpu.async_copy(h_hbm.at[sid4.at[0]], rows_v.at[0], gsem)

    def step(j, carry):
        par = lax.rem(j, 2)
        buf = rows_v.at[par]
        obuf = rows_v.at[1 - par]
        pltpu.make_async_copy(h_hbm.at[sid4.at[lax.rem(j, 4)]], buf,
                              gsem).wait()

        @pl.when(j > 0)
        def _():
            pltpu.make_async_copy(obuf, agg_sh.at[did4.at[lax.rem(j - 1, 4)]],
                                  ssem).wait()

        @pl.when(j + 2 < nb)
        def _():
            sl = lax.rem(j + 2, 4)
            _ring_copy(src_hbm.at[base + j + 2], sid4.at[sl], si0, si1,
                       par, False)
            _ring_copy(dst_hbm.at[base + j + 2], did4.at[sl], di0, di1,
                       par, False)

        @pl.when((j >= 1) & (j + 1 < nb))
        def _():
            sl = lax.rem(j + 1, 4)
            _ring_copy(src_hbm.at[base + j + 1], sid4.at[sl], si1, si0,
                       par, True)
            _ring_copy(dst_hbm.at[base + j + 1], did4.at[sl], di1, di0,
                       par, True)

        @pl.when(j + 1 < nb)
        def _():
            pltpu.async_copy(h_hbm.at[sid4.at[lax.rem(j + 1, 4)]], obuf, gsem)

        pltpu.async_copy(buf, agg_sh.at[did4.at[lax.rem(j, 4)]], ssem,
                         add=True)
        return carry

    lax.fori_loop(0, nb, step, 0)
    pltpu.make_async_copy(rows_v.at[(nb - 1) % 2],
                          agg_sh.at[did4.at[(nb - 1) % 4]], ssem).wait()


_MCH = 32


def _agg_body(nb0, nb1, d, h_hbm, src_hbm, dst_hbm, aggp_hbm,
              sid4, did4, rows_v, zbuf, agg_sh,
              gsem, ssem, si0, si1, di0, di1, msem):
    c = lax.axis_index("c")
    s = lax.axis_index("s")
    base = lax.select(c == 0, s * nb0, NS * nb0 + s * nb1)
    args = (h_hbm, src_hbm, dst_hbm, sid4, did4, rows_v, agg_sh,
            gsem, ssem, si0, si1, di0, di1)

    def work(nbc):
        for i in range(_MCH):
            for k in range(d // 16):
                zbuf[i, pl.ds(k * 16, 16)] = jnp.zeros((16,), jnp.float32)
        nfull = CHS // _MCH
        rem = CHS - nfull * _MCH
        for k in range(nfull):
            pltpu.async_copy(zbuf, agg_sh.at[pl.ds(s * CHS + k * _MCH,
                                                   _MCH)], msem)
        for k in range(nfull):
            pltpu.make_async_copy(zbuf, agg_sh.at[pl.ds(s * CHS + k * _MCH,
                                                        _MCH)], msem).wait()
        if rem:
            pltpu.sync_copy(zbuf.at[pl.ds(0, rem)],
                            agg_sh.at[pl.ds(s * CHS + nfull * _MCH, rem)])
        plsc.subcore_barrier()
        _agg_loop(nbc, base, *args)
        plsc.subcore_barrier()
        pltpu.sync_copy(agg_sh.at[pl.ds(s * CHS, CHS)],
                        aggp_hbm.at[c, pl.ds(s * CHS, CHS)])

    if nb0 > 0:
        @pl.when(c == 0)
        def _():
            work(nb0)

    if nb1 > 0:
        @pl.when(c == 1)
        def _():
            work(nb1)


def _make_agg(nb0, nb1, d):
    return pl.kernel(
        functools.partial(_agg_body, nb0, nb1, d),
        out_type=jax.ShapeDtypeStruct((NC, NPAD, d), jnp.float32),
        mesh=plsc.VectorSubcoreMesh(core_axis_name="c", subcore_axis_name="s"),
        scratch_types=[
            pltpu.VMEM((4, B), jnp.int32),
            pltpu.VMEM((4, B), jnp.int32),
            pltpu.VMEM((2, B, d), jnp.float32),
            pltpu.VMEM((_MCH, d), jnp.float32),
            pltpu.VMEM_SHARED((NSC, d), jnp.float32),
            pltpu.SemaphoreType.DMA,
            pltpu.SemaphoreType.DMA,
            pltpu.SemaphoreType.DMA,
            pltpu.SemaphoreType.DMA,
            pltpu.SemaphoreType.DMA,
            pltpu.SemaphoreType.DMA,
            pltpu.SemaphoreType.DMA,
        ],
    )


def _dinv(deg_blk):
    deg = deg_blk[:, 0:1] + deg_blk[:, 1:2] - 1.0
    return lax.rsqrt(jnp.maximum(deg, 1.0))


def _mm1_body(x_ref, w_ref, deg_ref, o_ref):
    di = _dinv(deg_ref[...])
    o_ref[...] = jnp.dot(x_ref[...], w_ref[...],
                         preferred_element_type=jnp.float32) * di


def _mm2_body(np_, *refs):
    ps, (h1_ref, deg_ref, b1_ref, w2_ref, o_ref) = refs[:np_], refs[np_:]
    di = _dinv(deg_ref[...])
    acc = h1_ref[...]
    for p in ps:
        acc = acc + p[...]
    t = acc * di + b1_ref[...]
    h = jnp.maximum(t, 0.0)
    o_ref[...] = jnp.dot(h, w2_ref[...],
                         preferred_element_type=jnp.float32) * di


def _out_body(np_, *refs):
    ps, (h2_ref, deg_ref, b2_ref, o_ref) = refs[:np_], refs[np_:]
    di = _dinv(deg_ref[...])
    acc = h2_ref[...]
    for q in ps:
        acc = acc + q[...]
    o_ref[...] = acc * di + b2_ref[...]


def _row_spec(d):
    return pl.BlockSpec((BR, d), lambda i: (i, 0))


_mm1 = pl.pallas_call(
    _mm1_body,
    grid=(NPAD // BR,),
    in_specs=[_row_spec(D_FEAT),
              pl.BlockSpec((D_FEAT, D_HID), lambda i: (0, 0)),
              _row_spec(128)],
    out_specs=_row_spec(D_HID),
    out_shape=jax.ShapeDtypeStruct((NPAD, D_HID), jnp.float32),
)

def _make_mm2(np_):
    return pl.pallas_call(
        functools.partial(_mm2_body, np_),
        grid=(NPAD // BR,),
        in_specs=[_row_spec(D_HID)] * np_ +
                 [_row_spec(D_HID), _row_spec(128),
                  pl.BlockSpec((1, D_HID), lambda i: (0, 0)),
                  pl.BlockSpec((D_HID, D2), lambda i: (0, 0))],
        out_specs=_row_spec(D2),
        out_shape=jax.ShapeDtypeStruct((NPAD, D2), jnp.float32),
    )


def _make_out(np_):
    return pl.pallas_call(
        functools.partial(_out_body, np_),
        grid=(NPAD // BR,),
        in_specs=[_row_spec(D2)] * np_ +
                 [_row_spec(D2), _row_spec(128),
                  pl.BlockSpec((1, D2), lambda i: (0, 0))],
        out_specs=_row_spec(D2),
        out_shape=jax.ShapeDtypeStruct((NPAD, D2), jnp.float32),
    )


NB0 = 152
NB1 = 8


def kernel(x, edge_index, W1, b1, W2, b2):
    n, e = x.shape[0], edge_index.shape[1]
    nb = -(-(-(-e // (NT * B))) // 8) * 8
    ep = NT * nb * B
    assert NS * (NB0 + NB1) * B == ep

    src = edge_index[0].astype(jnp.int32)
    dst = edge_index[1].astype(jnp.int32)
    pad = jnp.full((ep - e,), DUMMY, jnp.int32)
    src2 = jnp.concatenate([src, pad]).reshape(ep // B, B)
    dst2 = jnp.concatenate([dst, pad]).reshape(ep // B, B)

    x_pad = jnp.zeros((NPAD, D_FEAT), jnp.float32).at[:n].set(x)
    w2p = jnp.zeros((D_HID, D2), jnp.float32).at[:, :N_CLASSES].set(W2)
    b1r = b1.reshape(1, D_HID)
    b2p = jnp.zeros((1, D2), jnp.float32).at[0, :N_CLASSES].set(b2)

    ones_col = jnp.ones((NPAD,), jnp.float32)

    degp = _make_deg(nb)(dst2, ones_col).reshape(NC, NPAD, 1)
    degc = jnp.concatenate(
        [degp[0], degp[1], jnp.zeros((NPAD, 126), jnp.float32)], axis=1)

    np_ = 2 if NB1 > 0 else 1
    h1 = _mm1(x_pad, W1, degc)
    p = _make_agg(NB0, NB1, D_HID)(h1, src2, dst2)
    h2 = _make_mm2(np_)(*p[:np_], h1, degc, b1r, w2p)
    q = _make_agg(NB0, NB1, D2)(h2, src2, dst2)
    out = _make_out(np_)(*q[:np_], h2, degc, b2p)
    return out[:n, :N_CLASSES]


# speedup vs baseline: 1.0114x; 1.0114x over previous
import functools

import jax
import jax.numpy as jnp
from jax import lax
from jax.experimental import pallas as pl
from jax.experimental.pallas import tpu as pltpu
from jax.experimental.pallas import tpu_sc as plsc

N_NODES = 10000
D_FEAT = 128
D_HID = 128
N_CLASSES = 40

NC = 2
NS = 16
NT = NC * NS
B = 128
NPAD = 10240
DUMMY = N_NODES
CH = NPAD // NS
NSC = 10112
CHS = NSC // NS
D2 = 128
BR = 512


def _wid():
    return lax.axis_index("s") * NC + lax.axis_index("c")


def _deg_body(nb, dst_hbm, ones_hbm, degp_hbm, idx_v, ones_v, deg_sh, sem):
    del sem
    c = lax.axis_index("c")
    s = lax.axis_index("s")
    pltpu.sync_copy(ones_hbm.at[pl.ds(s * CH, CH)], deg_sh.at[pl.ds(s * CH, CH)])
    pltpu.sync_copy(dst_hbm.at[pl.ds(_wid() * nb, nb)], idx_v)
    pltpu.sync_copy(ones_hbm.at[pl.ds(0, B)], ones_v)
    plsc.subcore_barrier()

    def step(j, carry):
        pltpu.sync_copy(ones_v, deg_sh.at[idx_v.at[j]], add=True)
        return carry

    lax.fori_loop(0, nb, step, 0)
    plsc.subcore_barrier()
    pltpu.sync_copy(deg_sh.at[pl.ds(s * CH, CH)],
                    degp_hbm.at[pl.ds(c * NPAD + s * CH, CH)])


def _make_deg(nb):
    return pl.kernel(
        functools.partial(_deg_body, nb),
        out_type=jax.ShapeDtypeStruct((NC * NPAD,), jnp.float32),
        mesh=plsc.VectorSubcoreMesh(core_axis_name="c", subcore_axis_name="s"),
        scratch_types=[
            pltpu.VMEM((nb, B), jnp.int32),
            pltpu.VMEM((B,), jnp.float32),
            pltpu.VMEM_SHARED((NPAD,), jnp.float32),
            pltpu.SemaphoreType.DMA,
        ],
    )


def _ring_copy(row_hbm, slot, sem0, sem1, par, wait):
    @pl.when(par == 0)
    def _():
        if wait:
            pltpu.make_async_copy(row_hbm, slot, sem0).wait()
        else:
            pltpu.async_copy(row_hbm, slot, sem0)

    @pl.when(par == 1)
    def _():
        if wait:
            pltpu.make_async_copy(row_hbm, slot, sem1).wait()
        else:
            pltpu.async_copy(row_hbm, slot, sem1)


def _agg_loop(nb, base, h_hbm, src_hbm, dst_hbm, sid4, did4, rows_v, agg_sh,
              gsem, ssem, si0, si1, di0, di1):
    pltpu.sync_copy(src_hbm.at[base], sid4.at[0])
    pltpu.sync_copy(dst_hbm.at[base], did4.at[0])

    @pl.when(nb > 1)
    def _():
        pltpu.sync_copy(src_hbm.at[base + 1], sid4.at[1])
        pltpu.sync_copy(dst_hbm.at[base + 1], did4.at[1])

    pltpu.async_copy(h_hbm.at[sid4.at[0]], rows_v.at[0], gsem)

    def step(j, carry):
        par = lax.rem(j, 2)
        buf = rows_v.at[par]
        obuf = rows_v.at[1 - par]
        pltpu.make_async_copy(h_hbm.at[sid4.at[lax.rem(j, 4)]], buf,
                              gsem).wait()

        @pl.when(j > 0)
        def _():
            pltpu.make_async_copy(obuf, agg_sh.at[did4.at[lax.rem(j - 1, 4)]],
                                  ssem).wait()

        @pl.when(j + 2 < nb)
        def _():
            sl = lax.rem(j + 2, 4)
            _ring_copy(src_hbm.at[base + j + 2], sid4.at[sl], si0, si1,
                       par, False)
            _ring_copy(dst_hbm.at[base + j + 2], did4.at[sl], di0, di1,
                       par, False)

        @pl.when((j >= 1) & (j + 1 < nb))
        def _():
            sl = lax.rem(j + 1, 4)
            _ring_copy(src_hbm.at[base + j + 1], sid4.at[sl], si1, si0,
                       par, True)
            _ring_copy(dst_hbm.at[base + j + 1], did4.at[sl], di1, di0,
                       par, True)

        @pl.when(j + 1 < nb)
        def _():
            pltpu.async_copy(h_hbm.at[sid4.at[lax.rem(j + 1, 4)]], obuf, gsem)

        pltpu.async_copy(buf, agg_sh.at[did4.at[lax.rem(j, 4)]], ssem,
                         add=True)
        return carry

    lax.fori_loop(0, nb, step, 0)
    pltpu.make_async_copy(rows_v.at[(nb - 1) % 2],
                          agg_sh.at[did4.at[(nb - 1) % 4]], ssem).wait()


_MCH = 32


def _agg_body(nb0, nb1, d, h_hbm, src_hbm, dst_hbm, aggp_hbm,
              sid4, did4, rows_v, zbuf, agg_sh,
              gsem, ssem, si0, si1, di0, di1, msem):
    c = lax.axis_index("c")
    s = lax.axis_index("s")
    base = lax.select(c == 0, s * nb0, NS * nb0 + s * nb1)
    args = (h_hbm, src_hbm, dst_hbm, sid4, did4, rows_v, agg_sh,
            gsem, ssem, si0, si1, di0, di1)

    def work(nbc):
        for i in range(_MCH):
            for k in range(d // 16):
                zbuf[i, pl.ds(k * 16, 16)] = jnp.zeros((16,), jnp.float32)
        nfull = CHS // _MCH
        rem = CHS - nfull * _MCH
        for k in range(nfull):
            pltpu.async_copy(zbuf, agg_sh.at[pl.ds(s * CHS + k * _MCH,
                                                   _MCH)], msem)
        for k in range(nfull):
            pltpu.make_async_copy(zbuf, agg_sh.at[pl.ds(s * CHS + k * _MCH,
                                                        _MCH)], msem).wait()
        if rem:
            pltpu.sync_copy(zbuf.at[pl.ds(0, rem)],
                            agg_sh.at[pl.ds(s * CHS + nfull * _MCH, rem)])
        plsc.subcore_barrier()
        _agg_loop(nbc, base, *args)
        plsc.subcore_barrier()
        pltpu.sync_copy(agg_sh.at[pl.ds(s * CHS, CHS)],
                        aggp_hbm.at[c, pl.ds(s * CHS, CHS)])

    if nb0 > 0:
        @pl.when(c == 0)
        def _():
            work(nb0)

    if nb1 > 0:
        @pl.when(c == 1)
        def _():
            work(nb1)


def _make_agg(nb0, nb1, d):
    return pl.kernel(
        functools.partial(_agg_body, nb0, nb1, d),
        out_type=jax.ShapeDtypeStruct((NC, NPAD, d), jnp.float32),
        mesh=plsc.VectorSubcoreMesh(core_axis_name="c", subcore_axis_name="s"),
        scratch_types=[
            pltpu.VMEM((4, B), jnp.int32),
            pltpu.VMEM((4, B), jnp.int32),
            pltpu.VMEM((2, B, d), jnp.float32),
            pltpu.VMEM((_MCH, d), jnp.float32),
            pltpu.VMEM_SHARED((NSC, d), jnp.float32),
            pltpu.SemaphoreType.DMA,
            pltpu.SemaphoreType.DMA,
            pltpu.SemaphoreType.DMA,
            pltpu.SemaphoreType.DMA,
            pltpu.SemaphoreType.DMA,
            pltpu.SemaphoreType.DMA,
            pltpu.SemaphoreType.DMA,
        ],
    )


def _dinv(deg_blk):
    deg = deg_blk[:, 0:1] + deg_blk[:, 1:2] - 1.0
    return lax.rsqrt(jnp.maximum(deg, 1.0))


def _mm1_body(x_ref, w_ref, deg_ref, o_ref):
    di = _dinv(deg_ref[...])
    o_ref[...] = jnp.dot(x_ref[...], w_ref[...],
                         preferred_element_type=jnp.float32) * di


def _mm2_body(np_, *refs):
    ps, (h1_ref, deg_ref, b1_ref, w2_ref, o_ref) = refs[:np_], refs[np_:]
    di = _dinv(deg_ref[...])
    acc = h1_ref[...]
    for p in ps:
        acc = acc + p[...]
    t = acc * di + b1_ref[...]
    h = jnp.maximum(t, 0.0)
    o_ref[...] = jnp.dot(h, w2_ref[...],
                         preferred_element_type=jnp.float32) * di


def _out_body(np_, *refs):
    ps, (h2_ref, deg_ref, b2_ref, o_ref) = refs[:np_], refs[np_:]
    di = _dinv(deg_ref[...])
    acc = h2_ref[...]
    for q in ps:
        acc = acc + q[...]
    o_ref[...] = acc * di + b2_ref[...]


def _row_spec(d):
    return pl.BlockSpec((BR, d), lambda i: (i, 0))


_mm1 = pl.pallas_call(
    _mm1_body,
    grid=(NPAD // BR,),
    in_specs=[_row_spec(D_FEAT),
              pl.BlockSpec((D_FEAT, D_HID), lambda i: (0, 0)),
              _row_spec(128)],
    out_specs=_row_spec(D_HID),
    out_shape=jax.ShapeDtypeStruct((NPAD, D_HID), jnp.float32),
)

def _make_mm2(np_):
    return pl.pallas_call(
        functools.partial(_mm2_body, np_),
        grid=(NPAD // BR,),
        in_specs=[_row_spec(D_HID)] * np_ +
                 [_row_spec(D_HID), _row_spec(128),
                  pl.BlockSpec((1, D_HID), lambda i: (0, 0)),
                  pl.BlockSpec((D_HID, D2), lambda i: (0, 0))],
        out_specs=_row_spec(D2),
        out_shape=jax.ShapeDtypeStruct((NPAD, D2), jnp.float32),
    )


def _make_out(np_):
    return pl.pallas_call(
        functools.partial(_out_body, np_),
        grid=(NPAD // BR,),
        in_specs=[_row_spec(D2)] * np_ +
                 [_row_spec(D2), _row_spec(128),
                  pl.BlockSpec((1, D2), lambda i: (0, 0))],
        out_specs=_row_spec(D2),
        out_shape=jax.ShapeDtypeStruct((NPAD, D2), jnp.float32),
    )


NB0 = 144
NB1 = 16


def kernel(x, edge_index, W1, b1, W2, b2):
    n, e = x.shape[0], edge_index.shape[1]
    nb = -(-(-(-e // (NT * B))) // 8) * 8
    ep = NT * nb * B
    assert NS * (NB0 + NB1) * B == ep

    src = edge_index[0].astype(jnp.int32)
    dst = edge_index[1].astype(jnp.int32)
    pad = jnp.full((ep - e,), DUMMY, jnp.int32)
    src2 = jnp.concatenate([src, pad]).reshape(ep // B, B)
    dst2 = jnp.concatenate([dst, pad]).reshape(ep // B, B)

    x_pad = jnp.zeros((NPAD, D_FEAT), jnp.float32).at[:n].set(x)
    w2p = jnp.zeros((D_HID, D2), jnp.float32).at[:, :N_CLASSES].set(W2)
    b1r = b1.reshape(1, D_HID)
    b2p = jnp.zeros((1, D2), jnp.float32).at[0, :N_CLASSES].set(b2)

    ones_col = jnp.ones((NPAD,), jnp.float32)

    degp = _make_deg(nb)(dst2, ones_col).reshape(NC, NPAD, 1)
    degc = jnp.concatenate(
        [degp[0], degp[1], jnp.zeros((NPAD, 126), jnp.float32)], axis=1)

    np_ = 2 if NB1 > 0 else 1
    h1 = _mm1(x_pad, W1, degc)
    p = _make_agg(NB0, NB1, D_HID)(h1, src2, dst2)
    h2 = _make_mm2(np_)(*p[:np_], h1, degc, b1r, w2p)
    q = _make_agg(NB0, NB1, D2)(h2, src2, dst2)
    out = _make_out(np_)(*q[:np_], h2, degc, b2p)
    return out[:n, :N_CLASSES]
